# Initial kernel scaffold; baseline (speedup 1.0000x reference)
import jax
import jax.numpy as jnp
from jax.experimental import pallas as pl


def kernel(x, emb_table, fc_w, fc_b):
    raise NotImplementedError("write your pallas kernel here")


# speedup vs baseline: 8.6337x; 8.6337x over previous
import functools

import jax
import jax.numpy as jnp
from jax import lax
from jax.experimental import pallas as pl
from jax.experimental.pallas import tpu as pltpu
from jax.experimental.pallas import tpu_sc as plsc

PAD = 16
CHUNK = 100


def _proj_kernel(emb_ref, w_ref, b_ref, out_ref):
    out_ref[:, :] = (
        jnp.dot(
            emb_ref[:, :],
            w_ref[:, :],
            preferred_element_type=jnp.float32,
            precision=jax.lax.Precision.HIGHEST,
        )
        + b_ref[:, :]
    )


def _project_table(emb, w16, b16):
    v, e = emb.shape
    blk = 1000
    return pl.pallas_call(
        _proj_kernel,
        grid=(v // blk,),
        in_specs=[
            pl.BlockSpec((blk, e), lambda i: (i, 0)),
            pl.BlockSpec((e, PAD), lambda i: (0, 0)),
            pl.BlockSpec((1, PAD), lambda i: (0, 0)),
        ],
        out_specs=pl.BlockSpec((blk, PAD), lambda i: (i, 0)),
        out_shape=jax.ShapeDtypeStruct((v, PAD), jnp.float32),
    )(emb, w16, b16)


@functools.lru_cache(maxsize=None)
def _make_sc_pool(b_total, seq):
    info = plsc.get_sparse_core_info()
    nc, ns = info.num_cores, info.num_subcores
    nw = nc * ns
    nb = b_total // nw
    nchunk = seq // CHUNK
    nj = nb * nchunk
    mesh = plsc.VectorSubcoreMesh(core_axis_name="c", subcore_axis_name="s")

    @functools.partial(
        pl.kernel,
        mesh=mesh,
        compiler_params=pltpu.CompilerParams(use_tc_tiling_on_sc=False),
        out_type=jax.ShapeDtypeStruct((b_total, PAD), jnp.float32),
        scratch_types=[
            pltpu.VMEM((nj, CHUNK), jnp.int32),
            pltpu.VMEM((2, seq, PAD), jnp.float32),
            pltpu.VMEM((nb, PAD), jnp.float32),
            pltpu.SemaphoreType.DMA,
            pltpu.SemaphoreType.DMA,
        ],
    )
    def sc_pool(idx_hbm, p_hbm, out_hbm, idx_v, rows_v, acc_v, sem0, sem1):
        wid = lax.axis_index("s") * nc + lax.axis_index("c")
        sems = (sem0, sem1)
        pltpu.sync_copy(idx_hbm.at[pl.ds(wid * nj, nj), :], idx_v)

        def fire(e, buf):
            for h in range(nchunk):
                pltpu.async_copy(
                    p_hbm.at[idx_v.at[e * nchunk + h]],
                    rows_v.at[buf, pl.ds(h * CHUNK, CHUNK), :],
                    sems[buf],
                )

        def drain(e, buf):
            for h in range(nchunk):
                pltpu.make_async_copy(
                    p_hbm.at[idx_v.at[e * nchunk + h]],
                    rows_v.at[buf, pl.ds(h * CHUNK, CHUNK), :],
                    sems[buf],
                ).wait()

        def accumulate(e, buf):
            def inner(i, a):
                return a + rows_v[buf, i, :]

            acc = lax.fori_loop(0, seq, inner, jnp.zeros((PAD,), jnp.float32))
            acc_v[e, :] = acc

        fire(0, 0)

        def body(g, _):
            e = g * 2
            drain(e, 0)
            fire(e + 1, 1)
            accumulate(e, 0)
            drain(e + 1, 1)

            @pl.when(e + 2 < nb)
            def _():
                fire(e + 2, 0)

            accumulate(e + 1, 1)
            return 0

        lax.fori_loop(0, nb // 2, body, 0)
        pltpu.sync_copy(acc_v, out_hbm.at[pl.ds(wid * nb, nb), :])

    return sc_pool


def kernel(x, emb_table, fc_w, fc_b):
    b, s = x.shape
    v, e = emb_table.shape
    ncls = fc_w.shape[0]
    inv_s = 1.0 / s
    w16 = jnp.zeros((e, PAD), jnp.float32).at[:, :ncls].set(fc_w.T * inv_s)
    b16 = jnp.zeros((1, PAD), jnp.float32).at[0, :ncls].set(fc_b * inv_s)
    p = _project_table(emb_table, w16, b16)
    idx = x.astype(jnp.int32).reshape(b * (s // CHUNK), CHUNK)
    out16 = _make_sc_pool(b, s)(idx, p)
    return out16[:, :ncls]


# speedup vs baseline: 29.0338x; 3.3628x over previous
import functools

import jax
import jax.numpy as jnp
from jax import lax
from jax.experimental import pallas as pl
from jax.experimental.pallas import tpu as pltpu
from jax.experimental.pallas import tpu_sc as plsc

PAD = 16
NBUF = 8


def _proj_kernel(wt_ref, et_ref, b_ref, out_ref, t_ref):
    pt = (
        jnp.dot(
            wt_ref[:, :],
            et_ref[:, :],
            preferred_element_type=jnp.float32,
            precision=jax.lax.Precision.HIGHEST,
        )
        + b_ref[:, :]
    )
    t_ref[:, :] = pt.T
    blk8 = out_ref.shape[0]
    for k in range(8):
        out_ref[:, k * PAD : (k + 1) * PAD] = t_ref[pl.Slice(k, blk8, 8), :]


def _project_table_t(wt, et, bcol):
    e, v = et.shape
    blk = 12800
    return pl.pallas_call(
        _proj_kernel,
        grid=((v + blk - 1) // blk,),
        in_specs=[
            pl.BlockSpec((PAD, e), lambda i: (0, 0)),
            pl.BlockSpec((e, blk), lambda i: (0, i)),
            pl.BlockSpec((PAD, 1), lambda i: (0, 0)),
        ],
        out_specs=pl.BlockSpec((blk // 8, 8 * PAD), lambda i: (i, 0)),
        out_shape=jax.ShapeDtypeStruct((v // 8, 8 * PAD), jnp.float32),
        scratch_shapes=[pltpu.VMEM((blk, PAD), jnp.float32)],
    )(wt, et, bcol)


@functools.lru_cache(maxsize=None)
def _make_sc_pool(b_total, seq):
    info = plsc.get_sparse_core_info()
    nc, ns = info.num_cores, info.num_subcores
    nw = nc * ns
    nb = b_total // nw
    chunks = ((0, seq),)
    mesh = plsc.VectorSubcoreMesh(core_axis_name="c", subcore_axis_name="s")

    @functools.partial(
        pl.kernel,
        mesh=mesh,
        compiler_params=pltpu.CompilerParams(use_tc_tiling_on_sc=False),
        out_type=jax.ShapeDtypeStruct((b_total, PAD), jnp.float32),
        scratch_types=[
            pltpu.VMEM((nb, seq), jnp.int32),
            pltpu.VMEM((NBUF, seq, PAD), jnp.float32),
            pltpu.VMEM((nb, PAD), jnp.float32),
        ]
        + [pltpu.SemaphoreType.DMA] * NBUF,
    )
    def sc_pool(idx_hbm, p_hbm, out_hbm, idx_v, rows_v, acc_v, *sems):
        wid = lax.axis_index("s") * nc + lax.axis_index("c")
        pltpu.sync_copy(idx_hbm.at[pl.ds(wid * nb, nb), :], idx_v)

        def fire(e, buf):
            for off, n in chunks:
                pltpu.async_copy(
                    p_hbm.at[idx_v.at[e, pl.ds(off, n)]],
                    rows_v.at[buf, pl.ds(off, n), :],
                    sems[buf],
                )

        def drain(e, buf):
            for off, n in chunks:
                pltpu.make_async_copy(
                    p_hbm.at[idx_v.at[e, pl.ds(off, n)]],
                    rows_v.at[buf, pl.ds(off, n), :],
                    sems[buf],
                ).wait()

        def accumulate(e, buf):
            z = jnp.zeros((PAD,), jnp.float32)

            def inner(i, accs):
                a0, a1, a2, a3 = accs
                r = i * 4
                return (
                    a0 + rows_v[buf, r, :],
                    a1 + rows_v[buf, r + 1, :],
                    a2 + rows_v[buf, r + 2, :],
                    a3 + rows_v[buf, r + 3, :],
                )

            a0, a1, a2, a3 = lax.fori_loop(0, seq // 4, inner, (z, z, z, z))
            acc_v[e, :] = (a0 + a1) + (a2 + a3)

        for j in range(NBUF):
            fire(j, j)

        def body(g, _):
            for j in range(NBUF):
                e = g * NBUF + j
                drain(e, j)

                @pl.when(e + NBUF < nb)
                def _():
                    fire(e + NBUF, j)

                accumulate(e, j)
            return 0

        lax.fori_loop(0, nb // NBUF, body, 0)
        pltpu.sync_copy(acc_v, out_hbm.at[pl.ds(wid * nb, nb), :])

    return sc_pool


def kernel(x, emb_table, fc_w, fc_b):
    b, s = x.shape
    v, e = emb_table.shape
    ncls = fc_w.shape[0]
    inv_s = 1.0 / s
    wt = jnp.zeros((PAD, e), jnp.float32).at[:ncls, :].set(fc_w * inv_s)
    bcol = jnp.zeros((PAD, 1), jnp.float32).at[:ncls, 0].set(fc_b * inv_s)
    p8 = _project_table_t(wt, emb_table.T, bcol)
    p = p8.reshape(v, PAD)
    idx = x.astype(jnp.int32)
    out16 = _make_sc_pool(b, s)(idx, p)
    return out16[:, :ncls]
